# Initial kernel scaffold; baseline (speedup 1.0000x reference)
import jax
import jax.numpy as jnp
from jax.experimental import pallas as pl


def kernel(x, edge_index, edge_attr, batch, params):
    raise NotImplementedError("write your pallas kernel here")


# speedup vs baseline: 2.4321x; 2.4321x over previous
import functools

import jax
import jax.numpy as jnp
from jax import lax
from jax.experimental import pallas as pl
from jax.experimental.pallas import tpu as pltpu
from jax.experimental.pallas import tpu_sc as plsc

_N = 10000
_E = 320000
_D = 128
_DE = 16
_NG = 64

_NC = 2
_NS = 16
_NW = _NC * _NS
_N_PAD = 10240
_R = 2528
_E_PAD = _R * 128
_RPW = _R // _NW
_ROWS_PER_TILE = _N_PAD // _NS

_f32 = jnp.float32


def _enc_body(attr_ref, we_ref, be_ref, e0_ref, e1_ref, e2_ref):
    a = attr_ref[...]
    for l, ref in enumerate((e0_ref, e1_ref, e2_ref)):
        ref[...] = (
            jnp.dot(a, we_ref[l], preferred_element_type=_f32) + be_ref[l]
        )


def _encode(attr_p, we, be):
    blk = 1024
    grid = _E_PAD // blk
    return pl.pallas_call(
        _enc_body,
        grid=(grid,),
        in_specs=[
            pl.BlockSpec((blk, _DE), lambda i: (i, 0)),
            pl.BlockSpec((3, _DE, _D), lambda i: (0, 0, 0)),
            pl.BlockSpec((3, _D), lambda i: (0, 0)),
        ],
        out_specs=[pl.BlockSpec((blk, _D), lambda i: (i, 0))] * 3,
        out_shape=[jax.ShapeDtypeStruct((_E_PAD, _D), _f32)] * 3,
    )(attr_p, we, be)


def _sc_edge_body(h_hbm, e_hbm, src_hbm, dst_hbm, zero_hbm, a0_hbm, a1_hbm,
                  src_v, dst_v, e_v, agg_sh, sem):
    cid = lax.axis_index("c")
    sid = lax.axis_index("s")
    wid = sid * _NC + cid
    tile_rows = pl.ds(sid * _ROWS_PER_TILE, _ROWS_PER_TILE)

    pltpu.sync_copy(zero_hbm.at[tile_rows, :], agg_sh.at[tile_rows, :])
    plsc.subcore_barrier()

    def chunk(r, carry):
        row = wid * _RPW + r
        pltpu.sync_copy(src_hbm.at[row], src_v)
        pltpu.sync_copy(dst_hbm.at[row], dst_v)
        pltpu.sync_copy(e_hbm.at[row], e_v)
        pltpu.async_copy(h_hbm.at[src_v], e_v, sem, add=True).wait()

        def relu_row(j, c2):
            for dd in range(8):
                sl = pl.ds(dd * 16, 16)
                e_v[j, sl] = jnp.maximum(e_v[j, sl], 0.0)
            return c2

        lax.fori_loop(0, 128, relu_row, 0)
        pltpu.sync_copy(e_v, agg_sh.at[dst_v], add=True)
        return carry

    lax.fori_loop(0, _RPW, chunk, 0)
    plsc.subcore_barrier()

    @pl.when(cid == 0)
    def _():
        pltpu.sync_copy(agg_sh.at[tile_rows, :], a0_hbm.at[tile_rows, :])

    @pl.when(cid == 1)
    def _():
        pltpu.sync_copy(agg_sh.at[tile_rows, :], a1_hbm.at[tile_rows, :])


_edge_pass = pl.kernel(
    _sc_edge_body,
    out_type=(
        jax.ShapeDtypeStruct((_N_PAD, _D), _f32),
        jax.ShapeDtypeStruct((_N_PAD, _D), _f32),
    ),
    mesh=plsc.VectorSubcoreMesh(core_axis_name="c", subcore_axis_name="s"),
    scratch_types=[
        pltpu.VMEM((128,), jnp.int32),
        pltpu.VMEM((128,), jnp.int32),
        pltpu.VMEM((128, _D), _f32),
        pltpu.VMEM_SHARED((_N_PAD, _D), _f32),
        pltpu.SemaphoreType.DMA,
    ],
)


def _mlp_body(eps_ref, h_ref, a0_ref, a1_ref, w1_ref, b1_ref, w2_ref, b2_ref,
              o_ref):
    z = h_ref[...] * (1.0 + eps_ref[0, 0]) + a0_ref[...] + a1_ref[...]
    t = jnp.maximum(
        jnp.dot(z, w1_ref[...], preferred_element_type=_f32) + b1_ref[...],
        0.0,
    )
    o_ref[...] = jnp.dot(t, w2_ref[...], preferred_element_type=_f32) + b2_ref[...]


def _mlp(h, a0, a1, p):
    blk = 1024
    grid = _N_PAD // blk
    return pl.pallas_call(
        _mlp_body,
        grid=(grid,),
        in_specs=[
            pl.BlockSpec(memory_space=pltpu.SMEM),
            pl.BlockSpec((blk, _D), lambda i: (i, 0)),
            pl.BlockSpec((blk, _D), lambda i: (i, 0)),
            pl.BlockSpec((blk, _D), lambda i: (i, 0)),
            pl.BlockSpec((_D, 2 * _D), lambda i: (0, 0)),
            pl.BlockSpec((1, 2 * _D), lambda i: (0, 0)),
            pl.BlockSpec((2 * _D, _D), lambda i: (0, 0)),
            pl.BlockSpec((1, _D), lambda i: (0, 0)),
        ],
        out_specs=pl.BlockSpec((blk, _D), lambda i: (i, 0)),
        out_shape=jax.ShapeDtypeStruct((_N_PAD, _D), _f32),
    )(
        p["eps"].reshape(1, 1),
        h,
        a0,
        a1,
        p["W1"],
        p["b1"].reshape(1, 2 * _D),
        p["W2"],
        p["b2"].reshape(1, _D),
    )


def _pool_body(batch_ref, h_ref, wl1_ref, bl1_ref, wl2_ref, bl2_ref, o_ref):
    b = batch_ref[...]
    seg = lax.broadcasted_iota(jnp.int32, (_N_PAD, _NG), 1)
    mask = (b == seg).astype(_f32)
    g = lax.dot_general(
        mask, h_ref[...], (((0,), (0,)), ((), ())),
        preferred_element_type=_f32,
    )
    t = jnp.maximum(
        jnp.dot(g, wl1_ref[...], preferred_element_type=_f32) + bl1_ref[...],
        0.0,
    )
    o_ref[...] = jnp.dot(t, wl2_ref[...], preferred_element_type=_f32) + bl2_ref[...]


def _pool(h, batch_p, wl1, bl1, wl2, bl2):
    return pl.pallas_call(
        _pool_body,
        out_shape=jax.ShapeDtypeStruct((_NG, 1), _f32),
    )(batch_p, h, wl1, bl1.reshape(1, 64), wl2, bl2.reshape(1, 1))


@jax.jit
def kernel(x, edge_index, edge_attr, batch, params):
    src = edge_index[0]
    dst = edge_index[1]
    pad_e = _E_PAD - _E
    src_p = jnp.concatenate([src, jnp.zeros((pad_e,), jnp.int32)]).reshape(
        _R, 128)
    dst_p = jnp.concatenate([dst, jnp.full((pad_e,), _N, jnp.int32)]).reshape(
        _R, 128)
    attr_p = jnp.concatenate(
        [edge_attr, jnp.zeros((pad_e, _DE), _f32)], axis=0)
    h = jnp.concatenate([x, jnp.zeros((_N_PAD - _N, _D), _f32)], axis=0)
    zeros_nd = jnp.zeros((_N_PAD, _D), _f32)
    batch_p = jnp.concatenate(
        [batch, jnp.full((_N_PAD - _N,), _NG, jnp.int32)]).reshape(_N_PAD, 1)

    we = jnp.stack([p["We"] for p in params["convs"]])
    be = jnp.stack([p["be"] for p in params["convs"]])
    es = _encode(attr_p, we, be)

    for l, p in enumerate(params["convs"]):
        e3 = es[l].reshape(_R, 128, _D)
        a0, a1 = _edge_pass(h, e3, src_p, dst_p, zeros_nd)
        h = _mlp(h, a0, a1, p)

    return _pool(h, batch_p, params["Wl1"], params["bl1"], params["Wl2"],
                 params["bl2"])
